# baseline (speedup 1.0000x reference)
import functools

import jax
import jax.numpy as jnp
from jax import lax
from jax.experimental import pallas as pl
from jax.experimental.pallas import tpu as pltpu
from jax.experimental.pallas import tpu_sc as plsc

NC = 2
NS = 16
NW = NC * NS

BATCH = 16384
DIM = 32
CHUNK = 128
CHUNKS_PER_W = BATCH // (NW * CHUNK)


def _sc_gather4(u, i, user_gmf, item_gmf, user_mlp, item_mlp):
    u2 = u.reshape(BATCH // CHUNK, CHUNK)
    i2 = i.reshape(BATCH // CHUNK, CHUNK)
    out_t = jax.ShapeDtypeStruct((BATCH, DIM), jnp.float32)
    mesh = plsc.VectorSubcoreMesh(core_axis_name="c", subcore_axis_name="s")

    @functools.partial(
        pl.kernel,
        out_type=(out_t, out_t, out_t, out_t),
        mesh=mesh,
        compiler_params=pltpu.CompilerParams(use_tc_tiling_on_sc=False),
        scratch_types=[
            pltpu.VMEM((CHUNKS_PER_W, CHUNK), jnp.int32),
            pltpu.VMEM((CHUNKS_PER_W, CHUNK), jnp.int32),
            pltpu.VMEM((CHUNK, DIM), jnp.float32),
            pltpu.VMEM((CHUNK, DIM), jnp.float32),
            pltpu.VMEM((CHUNK, DIM), jnp.float32),
            pltpu.VMEM((CHUNK, DIM), jnp.float32),
            pltpu.SemaphoreType.DMA,
        ],
    )
    def k(ug_hbm, ig_hbm, um_hbm, im_hbm, u_hbm, i_hbm,
          oug, oig, oum, oim,
          uidx, iidx, r_ug, r_ig, r_um, r_im, sem):
        wid = lax.axis_index("s") * NC + lax.axis_index("c")
        row0 = wid * CHUNKS_PER_W
        pltpu.sync_copy(u_hbm.at[pl.ds(row0, CHUNKS_PER_W)], uidx)
        pltpu.sync_copy(i_hbm.at[pl.ds(row0, CHUNKS_PER_W)], iidx)
        for j in range(CHUNKS_PER_W):
            base = (row0 + j) * CHUNK
            c1 = pltpu.async_copy(ug_hbm.at[uidx.at[j]], r_ug, sem)
            c2 = pltpu.async_copy(ig_hbm.at[iidx.at[j]], r_ig, sem)
            c3 = pltpu.async_copy(um_hbm.at[uidx.at[j]], r_um, sem)
            c4 = pltpu.async_copy(im_hbm.at[iidx.at[j]], r_im, sem)
            c1.wait(); c2.wait(); c3.wait(); c4.wait()
            pltpu.sync_copy(r_ug, oug.at[pl.ds(base, CHUNK)])
            pltpu.sync_copy(r_ig, oig.at[pl.ds(base, CHUNK)])
            pltpu.sync_copy(r_um, oum.at[pl.ds(base, CHUNK)])
            pltpu.sync_copy(r_im, oim.at[pl.ds(base, CHUNK)])

    return k(user_gmf, item_gmf, user_mlp, item_mlp, u2, i2)


BLK = 2048


def _tc_body(ug_ref, ig_ref, um_ref, im_ref,
             w0u_ref, w0i_ref, b0_ref, w1_ref, b1_ref, w2_ref, b2_ref,
             whg_ref, whh_ref, bh_ref, o_ref):
    f32 = jnp.float32
    h = jnp.dot(um_ref[...], w0u_ref[...], preferred_element_type=f32)
    h = h + jnp.dot(im_ref[...], w0i_ref[...], preferred_element_type=f32)
    h = jnp.maximum(h + b0_ref[...], 0.0)
    h = jnp.maximum(jnp.dot(h, w1_ref[...], preferred_element_type=f32)
                    + b1_ref[...], 0.0)
    h = jnp.maximum(jnp.dot(h, w2_ref[...], preferred_element_type=f32)
                    + b2_ref[...], 0.0)
    gmf = ug_ref[...] * ig_ref[...]
    o_ref[...] = (jnp.dot(gmf, whg_ref[...], preferred_element_type=f32)
                  + jnp.dot(h, whh_ref[...], preferred_element_type=f32)
                  + bh_ref[...])


def _tc_dense(ug, ig, um, im, W0, b0, W1, b1, W2, b2, Wh, bh):
    w0u = W0[:, :DIM].T
    w0i = W0[:, DIM:].T
    w1 = W1.T
    w2 = W2.T
    whg = Wh[:, :DIM].T
    whh = Wh[:, DIM:].T
    b0r = b0.reshape(1, -1)
    b1r = b1.reshape(1, -1)
    b2r = b2.reshape(1, -1)
    bhr = bh.reshape(1, 1)

    n_blk = BATCH // BLK
    row_spec = pl.BlockSpec((BLK, DIM), lambda b: (b, 0))

    def w_spec(shape):
        return pl.BlockSpec(shape, lambda b: (0, 0))

    out = pl.pallas_call(
        _tc_body,
        grid=(n_blk,),
        in_specs=[
            row_spec, row_spec, row_spec, row_spec,
            w_spec(w0u.shape), w_spec(w0i.shape), w_spec(b0r.shape),
            w_spec(w1.shape), w_spec(b1r.shape),
            w_spec(w2.shape), w_spec(b2r.shape),
            w_spec(whg.shape), w_spec(whh.shape), w_spec(bhr.shape),
        ],
        out_specs=pl.BlockSpec((BLK, 1), lambda b: (b, 0)),
        out_shape=jax.ShapeDtypeStruct((BATCH, 1), jnp.float32),
    )(ug, ig, um, im, w0u, w0i, b0r, w1, b1r, w2, b2r, whg, whh, bhr)
    return out[:, 0]


def kernel(u, i, user_gmf, item_gmf, user_mlp, item_mlp,
           W0, b0, W1, b1, W2, b2, Wh, bh):
    ug, ig, um, im = _sc_gather4(u, i, user_gmf, item_gmf, user_mlp, item_mlp)
    return _tc_dense(ug, ig, um, im, W0, b0, W1, b1, W2, b2, Wh, bh)
